# baseline (speedup 1.0000x reference)
import functools

import jax
import jax.numpy as jnp
from jax.experimental import pallas as pl
from jax.experimental.pallas import tpu as pltpu


def _moe_body(xf_ref, xb_ref, gate_ref, wk_ref, bk_ref, wv_ref, bv_ref,
              wks_ref, bks_ref, wvs_ref, bvs_ref, out_ref, mask_ref, we_ref):
    e = pl.program_id(0)
    hb = pl.program_id(1)
    x = xb_ref[...]
    wk = wk_ref[0].astype(jnp.bfloat16)
    wv = wv_ref[0].astype(jnp.bfloat16)

    @pl.when((e == 0) & (hb == 0))
    def _init():
        logits = jnp.dot(xf_ref[...], gate_ref[...],
                         preferred_element_type=jnp.float32)
        ids = jax.lax.broadcasted_iota(jnp.int32, logits.shape, 1)
        a1 = jnp.argmax(logits, axis=1, keepdims=True)
        s1 = jnp.max(logits, axis=1, keepdims=True)
        masked = jnp.where(ids == a1, -jnp.inf, logits)
        a2 = jnp.argmax(masked, axis=1, keepdims=True)
        s2 = jnp.max(masked, axis=1, keepdims=True)
        e2 = jnp.exp(s2 - s1)
        w1 = 1.0 / (1.0 + e2)
        w2 = e2 / (1.0 + e2)
        mask_ref[...] = (jnp.where(ids == a1, w1, 0.0)
                         + jnp.where(ids == a2, w2, 0.0))
        out_ref[...] = jnp.zeros_like(out_ref)

    @pl.when(hb == 0)
    def _pick_w():
        ids = jax.lax.broadcasted_iota(jnp.int32, mask_ref.shape, 1)
        we_ref[...] = jnp.sum(mask_ref[...] * (ids == e), axis=1,
                              keepdims=True)

    w_e = we_ref[...]
    h = jax.nn.gelu(jnp.dot(x, wk, preferred_element_type=jnp.float32)
                    + bk_ref[0])
    out_ref[...] += jnp.dot((w_e * h).astype(jnp.bfloat16), wv,
                            preferred_element_type=jnp.float32)

    @pl.when(hb == 0)
    def _bias():
        out_ref[...] += w_e * bv_ref[0]

    @pl.when(e == 0)
    def _shared():
        hs = jax.nn.gelu(jnp.dot(x, wks_ref[...].astype(jnp.bfloat16),
                                 preferred_element_type=jnp.float32)
                         + bks_ref[...])
        out_ref[...] += jnp.dot(hs.astype(jnp.bfloat16),
                                wvs_ref[...].astype(jnp.bfloat16),
                                preferred_element_type=jnp.float32)

    @pl.when((e == 0) & (hb == 0))
    def _shared_bias():
        out_ref[...] += bvs_ref[...]


@functools.partial(jax.jit, static_argnames=())
def kernel(x, gate_kernel, Wk, bk, Wv, bv, Wk_s, bk_s, Wv_s, bv_s):
    B, S, D = x.shape
    T = B * S
    E = gate_kernel.shape[1]
    H = Wk.shape[2]
    HB = 512 if H % 512 == 0 else H
    NHB = H // HB

    x2 = x.reshape(T, D)
    xb = x2.astype(jnp.bfloat16)
    bk2 = bk.reshape(E, 1, H)
    bv2 = bv.reshape(E, 1, D)
    bks2 = bk_s.reshape(1, H)
    bvs2 = bv_s.reshape(1, D)

    out = pl.pallas_call(
        _moe_body,
        grid=(E, NHB),
        in_specs=[
            pl.BlockSpec((T, D), lambda e, h: (0, 0)),
            pl.BlockSpec((T, D), lambda e, h: (0, 0)),
            pl.BlockSpec((D, E), lambda e, h: (0, 0)),
            pl.BlockSpec((1, D, HB), lambda e, h: (e, 0, h)),
            pl.BlockSpec((1, 1, HB), lambda e, h: (e, 0, h)),
            pl.BlockSpec((1, HB, D), lambda e, h: (e, h, 0)),
            pl.BlockSpec((1, 1, D), lambda e, h: (e, 0, 0)),
            pl.BlockSpec((D, HB), lambda e, h: (0, h)),
            pl.BlockSpec((1, HB), lambda e, h: (0, h)),
            pl.BlockSpec((HB, D), lambda e, h: (h, 0)),
            pl.BlockSpec((1, D), lambda e, h: (0, 0)),
        ],
        out_specs=pl.BlockSpec((T, D), lambda e, h: (0, 0)),
        out_shape=jax.ShapeDtypeStruct((T, D), jnp.float32),
        scratch_shapes=[pltpu.VMEM((T, E), jnp.float32),
                        pltpu.VMEM((T, 1), jnp.float32)],
        compiler_params=pltpu.CompilerParams(
            dimension_semantics=("arbitrary", "arbitrary")),
    )(x2, xb, gate_kernel, Wk, bk2, Wv, bv2, Wk_s, bks2, Wv_s, bvs2)

    return (out.reshape(B, S, D), jnp.float32(0.0))


# speedup vs baseline: 1.0984x; 1.0984x over previous
import functools

import jax
import jax.numpy as jnp
from jax.experimental import pallas as pl
from jax.experimental.pallas import tpu as pltpu

_TB = 256
_HB = 512


def _router_body(x_ref, gate_ref,
                 p1_ref, p2_ref, w1_ref, w2_ref, gid_ref, valid_ref,
                 *, E, T, NB):
    logits = jnp.dot(x_ref[...], gate_ref[...],
                     preferred_element_type=jnp.float32)
    ids8 = jax.lax.broadcasted_iota(jnp.int32, (T, E), 1)
    a1 = jnp.argmax(logits, axis=1, keepdims=True)
    s1 = jnp.max(logits, axis=1, keepdims=True)
    masked = jnp.where(ids8 == a1, -jnp.inf, logits)
    a2 = jnp.argmax(masked, axis=1, keepdims=True)
    s2 = jnp.max(masked, axis=1, keepdims=True)
    e2 = jnp.exp(s2 - s1)
    w1_ref[...] = 1.0 / (1.0 + e2)
    w2_ref[...] = e2 / (1.0 + e2)

    oh1 = (ids8 == a1)
    oh2 = (ids8 == a2)
    oh1f = oh1.astype(jnp.float32)
    oh2f = oh2.astype(jnp.float32)

    tri = (jax.lax.broadcasted_iota(jnp.int32, (T, T), 0)
           > jax.lax.broadcasted_iota(jnp.int32, (T, T), 1)
           ).astype(jnp.bfloat16)
    s1cnt = jnp.dot(tri, oh1.astype(jnp.bfloat16),
                    preferred_element_type=jnp.float32)
    s2cnt = jnp.dot(tri, oh2.astype(jnp.bfloat16),
                    preferred_element_type=jnp.float32)
    c0 = jnp.sum(oh1f, axis=0, keepdims=True)
    c1 = jnp.sum(oh2f, axis=0, keepdims=True)
    c = c0 + c1
    pc = jnp.ceil(c / _TB) * _TB
    triu8 = (jax.lax.broadcasted_iota(jnp.int32, (E, E), 0)
             < jax.lax.broadcasted_iota(jnp.int32, (E, E), 1)
             ).astype(jnp.float32)
    offs = jnp.dot(pc, triu8, preferred_element_type=jnp.float32)

    rank1 = jnp.sum(oh1f * s1cnt, axis=1, keepdims=True)
    rank2 = jnp.sum(oh2f * (s2cnt + c0), axis=1, keepdims=True)
    off1 = jnp.sum(oh1f * offs, axis=1, keepdims=True)
    off2 = jnp.sum(oh2f * offs, axis=1, keepdims=True)
    p1_ref[...] = (off1 + rank1).astype(jnp.int32)
    p2_ref[...] = (off2 + rank2).astype(jnp.int32)

    sb = (jax.lax.broadcasted_iota(jnp.int32, (64, 1), 0)
          .astype(jnp.float32) * _TB)
    gid = jnp.sum((offs <= sb).astype(jnp.float32), axis=1,
                  keepdims=True) - 1.0
    total = jnp.sum(pc, axis=1, keepdims=True)
    gid_ref[...] = jnp.clip(gid, 0.0, E - 1.0).astype(jnp.int32)
    valid_ref[...] = (sb < total).astype(jnp.int32)


def _expert_body(gid_sref, valid_sref, xb_ref, p1_ref, p2_ref,
                 wk_ref, bk_ref, wv_ref, bv_ref, y_ref, *, T, D, H):
    b = pl.program_id(0)

    @pl.when(valid_sref[b] == 1)
    def _compute():
        slot = (jax.lax.broadcasted_iota(jnp.int32, (_TB, T), 0)
                + b * _TB)
        g = ((p1_ref[...] == slot) | (p2_ref[...] == slot)
             ).astype(jnp.bfloat16)
        xs = jnp.dot(g, xb_ref[...],
                     preferred_element_type=jnp.float32).astype(jnp.bfloat16)
        acc = jnp.zeros((_TB, D), dtype=jnp.float32)
        for c in range(H // _HB):
            sl = slice(c * _HB, (c + 1) * _HB)
            h = jax.nn.gelu(
                jnp.dot(xs, wk_ref[0][:, sl].astype(jnp.bfloat16),
                        preferred_element_type=jnp.float32)
                + bk_ref[0, :, sl])
            acc = acc + jnp.dot(h.astype(jnp.bfloat16),
                                wv_ref[0][sl, :].astype(jnp.bfloat16),
                                preferred_element_type=jnp.float32)
        y_ref[...] = acc + bv_ref[0]

    @pl.when(valid_sref[b] == 0)
    def _pad():
        y_ref[...] = jnp.zeros_like(y_ref)


def _combine_body(xb_ref, p1_ref, p2_ref, w1_ref, w2_ref, y_ref,
                  wks_ref, bks_ref, wvs_ref, bvs_ref, out_ref, *, NS):
    hb = pl.program_id(1)

    @pl.when(hb == 0)
    def _combine():
        s_ids = jax.lax.broadcasted_iota(jnp.int32, (out_ref.shape[0], NS), 1)
        cb = (jnp.where(p1_ref[...] == s_ids, w1_ref[...], 0.0)
              + jnp.where(p2_ref[...] == s_ids, w2_ref[...], 0.0))
        out_ref[...] = (jnp.dot(cb, y_ref[...],
                                preferred_element_type=jnp.float32)
                        + bvs_ref[...])

    hs = jax.nn.gelu(jnp.dot(xb_ref[...],
                             wks_ref[...].astype(jnp.bfloat16),
                             preferred_element_type=jnp.float32)
                     + bks_ref[...])
    out_ref[...] += jnp.dot(hs.astype(jnp.bfloat16),
                            wvs_ref[...].astype(jnp.bfloat16),
                            preferred_element_type=jnp.float32)


@functools.partial(jax.jit, static_argnames=())
def kernel(x, gate_kernel, Wk, bk, Wv, bv, Wk_s, bk_s, Wv_s, bv_s):
    B, S, D = x.shape
    T = B * S
    E = gate_kernel.shape[1]
    H = Wk.shape[2]
    NB = (2 * T) // _TB + E
    NS = NB * _TB
    NHB = H // _HB

    x2 = x.reshape(T, D)
    xb = x2.astype(jnp.bfloat16)
    bk3 = bk.reshape(E, 1, H)
    bv3 = bv.reshape(E, 1, D)
    bks2 = bk_s.reshape(1, H)
    bvs2 = bv_s.reshape(1, D)

    router = pl.pallas_call(
        functools.partial(_router_body, E=E, T=T, NB=NB),
        in_specs=[pl.BlockSpec((T, D), lambda: (0, 0)),
                  pl.BlockSpec((D, E), lambda: (0, 0))],
        out_specs=[pl.BlockSpec((T, 1), lambda: (0, 0)),
                   pl.BlockSpec((T, 1), lambda: (0, 0)),
                   pl.BlockSpec((T, 1), lambda: (0, 0)),
                   pl.BlockSpec((T, 1), lambda: (0, 0)),
                   pl.BlockSpec((64, 1), lambda: (0, 0)),
                   pl.BlockSpec((64, 1), lambda: (0, 0))],
        out_shape=[jax.ShapeDtypeStruct((T, 1), jnp.int32),
                   jax.ShapeDtypeStruct((T, 1), jnp.int32),
                   jax.ShapeDtypeStruct((T, 1), jnp.float32),
                   jax.ShapeDtypeStruct((T, 1), jnp.float32),
                   jax.ShapeDtypeStruct((64, 1), jnp.int32),
                   jax.ShapeDtypeStruct((64, 1), jnp.int32)],
    )(x2, gate_kernel)
    p1c, p2c, w1c, w2c, gid64, valid64 = router
    p1r = p1c.reshape(1, T)
    p2r = p2c.reshape(1, T)
    gids = gid64.reshape(64)[:NB]
    valid = valid64.reshape(64)[:NB]

    y = pl.pallas_call(
        functools.partial(_expert_body, T=T, D=D, H=H),
        grid_spec=pltpu.PrefetchScalarGridSpec(
            num_scalar_prefetch=2,
            grid=(NB,),
            in_specs=[
                pl.BlockSpec((T, D), lambda b, g, v: (0, 0)),
                pl.BlockSpec((1, T), lambda b, g, v: (0, 0)),
                pl.BlockSpec((1, T), lambda b, g, v: (0, 0)),
                pl.BlockSpec((1, D, H), lambda b, g, v: (g[b], 0, 0)),
                pl.BlockSpec((1, 1, H), lambda b, g, v: (g[b], 0, 0)),
                pl.BlockSpec((1, H, D), lambda b, g, v: (g[b], 0, 0)),
                pl.BlockSpec((1, 1, D), lambda b, g, v: (g[b], 0, 0)),
            ],
            out_specs=pl.BlockSpec((_TB, D), lambda b, g, v: (b, 0)),
        ),
        out_shape=jax.ShapeDtypeStruct((NS, D), jnp.float32),
        compiler_params=pltpu.CompilerParams(
            dimension_semantics=("arbitrary",)),
    )(gids, valid, xb, p1r, p2r, Wk, bk3, Wv, bv3)

    TBC = 256
    out = pl.pallas_call(
        functools.partial(_combine_body, NS=NS),
        grid=(T // TBC, NHB),
        in_specs=[
            pl.BlockSpec((TBC, D), lambda t, h: (t, 0)),
            pl.BlockSpec((TBC, 1), lambda t, h: (t, 0)),
            pl.BlockSpec((TBC, 1), lambda t, h: (t, 0)),
            pl.BlockSpec((TBC, 1), lambda t, h: (t, 0)),
            pl.BlockSpec((TBC, 1), lambda t, h: (t, 0)),
            pl.BlockSpec((NS, D), lambda t, h: (0, 0)),
            pl.BlockSpec((D, _HB), lambda t, h: (0, h)),
            pl.BlockSpec((1, _HB), lambda t, h: (0, h)),
            pl.BlockSpec((_HB, D), lambda t, h: (h, 0)),
            pl.BlockSpec((1, D), lambda t, h: (0, 0)),
        ],
        out_specs=pl.BlockSpec((TBC, D), lambda t, h: (t, 0)),
        out_shape=jax.ShapeDtypeStruct((T, D), jnp.float32),
        compiler_params=pltpu.CompilerParams(
            dimension_semantics=("arbitrary", "arbitrary")),
    )(xb, p1c, p2c, w1c, w2c, y, Wk_s, bks2, Wv_s, bvs2)

    return (out.reshape(B, S, D), jnp.float32(0.0))
